# baseline (speedup 1.0000x reference)
import functools

import jax
import jax.numpy as jnp
from jax import lax
from jax.experimental import pallas as pl
from jax.experimental.pallas import tpu as pltpu
from jax.experimental.pallas import tpu_sc as plsc

B_, L_, D_, H_, E_ = 2, 8192, 768, 768, 64
S_ = B_ * L_
CAP = 256
ZROW = E_ * CAP
RPAD = (E_ + 1) * CAP
TB = 256
NB = S_ // TB
NW = 32
TW = S_ // NW
CH = 32
NCH = TW // CH
NBUF = 4
GW = 128
CNBUF = 5


def _gating_body(x_ref, wg_ref, pos_ref, gate_ref, cnt_ref, laux_ref,
                 cnt_acc, me_acc):
    i = pl.program_id(0)

    @pl.when(i == 0)
    def _():
        cnt_acc[...] = jnp.zeros_like(cnt_acc)
        me_acc[...] = jnp.zeros_like(me_acc)

    x = x_ref[...]
    logits = jnp.dot(x, wg_ref[...], preferred_element_type=jnp.float32)
    m = jnp.max(logits, axis=-1, keepdims=True)
    ex = jnp.exp(logits - m)
    gates = ex / jnp.sum(ex, axis=-1, keepdims=True)
    idx = jnp.argmax(gates, axis=-1).astype(jnp.int32)
    gmax = jnp.max(gates, axis=-1)

    eiota = lax.broadcasted_iota(jnp.int32, (TB, E_), 1)
    mask1 = (eiota == idx[:, None]).astype(jnp.float32)

    riota = lax.broadcasted_iota(jnp.int32, (TB, TB), 0)
    ciota = lax.broadcasted_iota(jnp.int32, (TB, TB), 1)
    tri = (ciota <= riota).astype(jnp.bfloat16)
    incl = jnp.dot(tri, mask1.astype(jnp.bfloat16),
                   preferred_element_type=jnp.float32)

    prior = cnt_acc[...]
    loc = jnp.sum((incl - 1.0 + prior) * mask1, axis=1)
    within = loc < float(CAP)
    locc = jnp.minimum(loc, float(CAP - 1)).astype(jnp.int32)
    pos = idx * CAP + locc
    pos_ref[...] = jnp.where(within, pos, ZROW)
    gate_ref[...] = jnp.broadcast_to(gmax[:, None], (TB, GW))

    cnt_new = prior + jnp.sum(mask1, axis=0, keepdims=True)
    cnt_acc[...] = cnt_new
    me_new = me_acc[...] + jnp.sum(gates, axis=0, keepdims=True)
    me_acc[...] = me_new

    @pl.when(i == NB - 1)
    def _():
        cnt_ref[...] = cnt_new[0].astype(jnp.int32)
        me = me_new / float(S_)
        ce = cnt_new / float(S_)
        laux_ref[...] = jnp.sum(me * ce).reshape(1, 1) * float(E_)


def _gating(x, wg):
    return pl.pallas_call(
        _gating_body,
        grid=(NB,),
        in_specs=[
            pl.BlockSpec((TB, D_), lambda i: (i, 0)),
            pl.BlockSpec((D_, E_), lambda i: (0, 0)),
        ],
        out_specs=[
            pl.BlockSpec((TB,), lambda i: (i,)),
            pl.BlockSpec((TB, GW), lambda i: (i, 0)),
            pl.BlockSpec((E_,), lambda i: (0,)),
            pl.BlockSpec((1, 1), lambda i: (0, 0)),
        ],
        out_shape=[
            jax.ShapeDtypeStruct((S_,), jnp.int32),
            jax.ShapeDtypeStruct((S_, GW), jnp.float32),
            jax.ShapeDtypeStruct((E_,), jnp.int32),
            jax.ShapeDtypeStruct((1, 1), jnp.float32),
        ],
        scratch_shapes=[
            pltpu.VMEM((1, E_), jnp.float32),
            pltpu.VMEM((1, E_), jnp.float32),
        ],
        compiler_params=pltpu.CompilerParams(
            dimension_semantics=("arbitrary",)),
    )(x, wg)


def _dispatch_body(x_hbm, pos_hbm, gate_hbm, disp_hbm, scale_hbm,
                   idx_v, gb0, gb1, gb2, gb3, xb0, xb1, xb2, xb3,
                   sem_in, sem_sc):
    nc = plsc.get_sparse_core_info().num_cores
    wid = lax.axis_index("s") * nc + lax.axis_index("c")
    base = wid * TW
    pltpu.sync_copy(pos_hbm.at[wid], idx_v)

    xb = [xb0, xb1, xb2, xb3]
    gb = [gb0, gb1, gb2, gb3]
    LOOK = NBUF - 2

    def start_in(j):
        b = j % NBUF
        return (
            pltpu.async_copy(x_hbm.at[pl.ds(base + CH * j, CH)], xb[b], sem_in),
            pltpu.async_copy(gate_hbm.at[pl.ds(base + CH * j, CH)], gb[b], sem_in),
        )

    hin = {j: start_in(j) for j in range(min(LOOK + 1, NCH))}
    hsc = {}
    sc_waited = 0
    for j in range(NCH):
        b = j % NBUF
        for h in hin[j]:
            h.wait()
        hsc[j] = (
            pltpu.async_copy(xb[b], disp_hbm.at[idx_v.at[j]], sem_sc),
            pltpu.async_copy(gb[b], scale_hbm.at[idx_v.at[j]], sem_sc),
        )
        jn = j + LOOK + 1
        if jn < NCH:
            while sc_waited < jn - NBUF + 1:
                for h in hsc[sc_waited]:
                    h.wait()
                sc_waited += 1
            hin[jn] = start_in(jn)
    while sc_waited < NCH:
        for h in hsc[sc_waited]:
            h.wait()
        sc_waited += 1


def _dispatch(x, pos3, gate):
    mesh = plsc.VectorSubcoreMesh(core_axis_name="c", subcore_axis_name="s")
    fn = functools.partial(
        pl.kernel,
        mesh=mesh,
        out_type=[
            jax.ShapeDtypeStruct((RPAD, D_), jnp.float32),
            jax.ShapeDtypeStruct((RPAD, GW), jnp.float32),
        ],
        scratch_types=(
            [pltpu.VMEM((NCH, CH), jnp.int32)]
            + [pltpu.VMEM((CH, GW), jnp.float32) for _ in range(NBUF)]
            + [pltpu.VMEM((CH, D_), jnp.float32) for _ in range(NBUF)]
            + [pltpu.SemaphoreType.DMA, pltpu.SemaphoreType.DMA]
        ),
    )(_dispatch_body)
    return fn(x, pos3, gate)


def _mlp_body(d_ref, w1_ref, b1_ref, w2_ref, b2_ref, s_ref, o_ref):
    e = pl.program_id(0)

    @pl.when(e < E_)
    def _():
        d = d_ref[...].astype(jnp.bfloat16)
        h = jnp.dot(d, w1_ref[0].astype(jnp.bfloat16),
                    preferred_element_type=jnp.float32)
        h = jnp.maximum(h + b1_ref[0], 0.0)
        o = jnp.dot(h.astype(jnp.bfloat16), w2_ref[0].astype(jnp.bfloat16),
                    preferred_element_type=jnp.float32)
        o = o + b2_ref[0]
        o_ref[...] = o * s_ref[...][:, 0:1]

    @pl.when(e == E_)
    def _():
        o_ref[...] = jnp.zeros_like(o_ref)


def _mlp(disp, w1, b1, w2, b2, scale):
    ew3 = lambda e: (jnp.minimum(e, E_ - 1), 0, 0)
    return pl.pallas_call(
        _mlp_body,
        grid=(E_ + 1,),
        in_specs=[
            pl.BlockSpec((CAP, D_), lambda e: (e, 0)),
            pl.BlockSpec((1, D_, H_), ew3),
            pl.BlockSpec((1, 1, H_), ew3),
            pl.BlockSpec((1, H_, D_), ew3),
            pl.BlockSpec((1, 1, D_), ew3),
            pl.BlockSpec((CAP, GW), lambda e: (e, 0)),
        ],
        out_specs=pl.BlockSpec((CAP, D_), lambda e: (e, 0)),
        out_shape=jax.ShapeDtypeStruct((RPAD, D_), jnp.float32),
        compiler_params=pltpu.CompilerParams(
            dimension_semantics=("arbitrary",)),
    )(disp, w1, b1.reshape(E_, 1, H_), w2, b2.reshape(E_, 1, D_), scale)


def _combine_body(eo_hbm, pos_hbm, out_hbm, idx_v, buf0, buf1, buf2, buf3,
                  buf4, sem_g, sem_o):
    nc = plsc.get_sparse_core_info().num_cores
    wid = lax.axis_index("s") * nc + lax.axis_index("c")
    base = wid * TW
    pltpu.sync_copy(pos_hbm.at[wid], idx_v)

    buf = [buf0, buf1, buf2, buf3, buf4]
    LOOK = CNBUF - 2

    def start_g(j):
        return pltpu.async_copy(eo_hbm.at[idx_v.at[j]], buf[j % CNBUF], sem_g)

    hg = {j: start_g(j) for j in range(min(LOOK + 1, NCH))}
    ho = {}
    o_waited = 0
    for j in range(NCH):
        b = j % CNBUF
        hg[j].wait()
        ho[j] = pltpu.async_copy(buf[b], out_hbm.at[pl.ds(base + CH * j, CH)],
                                 sem_o)
        jn = j + LOOK + 1
        if jn < NCH:
            while o_waited < jn - CNBUF + 1:
                ho[o_waited].wait()
                o_waited += 1
            hg[jn] = start_g(jn)
    while o_waited < NCH:
        ho[o_waited].wait()
        o_waited += 1


def _combine(eo, pos3):
    mesh = plsc.VectorSubcoreMesh(core_axis_name="c", subcore_axis_name="s")
    fn = functools.partial(
        pl.kernel,
        mesh=mesh,
        out_type=jax.ShapeDtypeStruct((S_, D_), jnp.float32),
        scratch_types=(
            [pltpu.VMEM((NCH, CH), jnp.int32)]
            + [pltpu.VMEM((CH, D_), jnp.float32) for _ in range(CNBUF)]
            + [pltpu.SemaphoreType.DMA, pltpu.SemaphoreType.DMA]
        ),
    )(_combine_body)
    return fn(eo, pos3)


def kernel(hidden_states, wg, w1, b1, w2, b2):
    x = hidden_states.reshape(S_, D_)
    pos, gate, counts, laux = _gating(x, wg)
    pos3 = pos.reshape(NW, NCH, CH)
    disp, scale = _dispatch(x, pos3, gate)
    eo = _mlp(disp, w1, b1, w2, b2, scale)
    out = _combine(eo, pos3)
    return out.reshape(B_, L_, D_), laux.reshape(()), counts


# speedup vs baseline: 1.0577x; 1.0577x over previous
import functools

import jax
import jax.numpy as jnp
from jax import lax
from jax.experimental import pallas as pl
from jax.experimental.pallas import tpu as pltpu
from jax.experimental.pallas import tpu_sc as plsc

B_, L_, D_, H_, E_ = 2, 8192, 768, 768, 64
S_ = B_ * L_
CAP = 256
ZROW = E_ * CAP
RPAD = (E_ + 1) * CAP
TB = 1024
NB = S_ // TB
NW = 32
TW = S_ // NW
CH = 32
NCH = TW // CH
NBUF = 4
GW = 128
CNBUF = 5


def _gating_body(x_ref, wg_ref, pos_ref, gate_ref, cnt_ref, laux_ref,
                 cnt_acc, me_acc):
    i = pl.program_id(0)

    @pl.when(i == 0)
    def _():
        cnt_acc[...] = jnp.zeros_like(cnt_acc)
        me_acc[...] = jnp.zeros_like(me_acc)

    x = x_ref[...]
    logits = jnp.dot(x, wg_ref[...], preferred_element_type=jnp.float32)
    m = jnp.max(logits, axis=-1, keepdims=True)
    ex = jnp.exp(logits - m)
    gates = ex / jnp.sum(ex, axis=-1, keepdims=True)
    idx = jnp.argmax(gates, axis=-1).astype(jnp.int32)
    gmax = jnp.max(gates, axis=-1)

    eiota = lax.broadcasted_iota(jnp.int32, (TB, E_), 1)
    mask1 = (eiota == idx[:, None]).astype(jnp.float32)

    riota = lax.broadcasted_iota(jnp.int32, (TB, TB), 0)
    ciota = lax.broadcasted_iota(jnp.int32, (TB, TB), 1)
    tri = (ciota <= riota).astype(jnp.bfloat16)
    incl = jnp.dot(tri, mask1.astype(jnp.bfloat16),
                   preferred_element_type=jnp.float32)

    prior = cnt_acc[...]
    loc = jnp.sum((incl - 1.0 + prior) * mask1, axis=1)
    within = loc < float(CAP)
    locc = jnp.minimum(loc, float(CAP - 1)).astype(jnp.int32)
    pos = idx * CAP + locc
    pos_ref[...] = jnp.where(within, pos, ZROW)
    gate_ref[...] = jnp.broadcast_to(gmax[:, None], (TB, GW))

    cnt_new = prior + jnp.sum(mask1, axis=0, keepdims=True)
    cnt_acc[...] = cnt_new
    me_new = me_acc[...] + jnp.sum(gates, axis=0, keepdims=True)
    me_acc[...] = me_new

    @pl.when(i == NB - 1)
    def _():
        cnt_ref[...] = cnt_new[0].astype(jnp.int32)
        me = me_new / float(S_)
        ce = cnt_new / float(S_)
        laux_ref[...] = jnp.sum(me * ce).reshape(1, 1) * float(E_)


def _gating(x, wg):
    return pl.pallas_call(
        _gating_body,
        grid=(NB,),
        in_specs=[
            pl.BlockSpec((TB, D_), lambda i: (i, 0)),
            pl.BlockSpec((D_, E_), lambda i: (0, 0)),
        ],
        out_specs=[
            pl.BlockSpec((TB,), lambda i: (i,)),
            pl.BlockSpec((TB, GW), lambda i: (i, 0)),
            pl.BlockSpec((E_,), lambda i: (0,)),
            pl.BlockSpec((1, 1), lambda i: (0, 0)),
        ],
        out_shape=[
            jax.ShapeDtypeStruct((S_,), jnp.int32),
            jax.ShapeDtypeStruct((S_, GW), jnp.float32),
            jax.ShapeDtypeStruct((E_,), jnp.int32),
            jax.ShapeDtypeStruct((1, 1), jnp.float32),
        ],
        scratch_shapes=[
            pltpu.VMEM((1, E_), jnp.float32),
            pltpu.VMEM((1, E_), jnp.float32),
        ],
        compiler_params=pltpu.CompilerParams(
            dimension_semantics=("arbitrary",)),
    )(x, wg)


def _dispatch_body(x_hbm, pos_hbm, gate_hbm, disp_hbm, scale_hbm,
                   idx_v, gb0, gb1, gb2, gb3, xb0, xb1, xb2, xb3,
                   sem_in, sem_sc):
    nc = plsc.get_sparse_core_info().num_cores
    wid = lax.axis_index("s") * nc + lax.axis_index("c")
    base = wid * TW
    pltpu.sync_copy(pos_hbm.at[wid], idx_v)

    xb = [xb0, xb1, xb2, xb3]
    gb = [gb0, gb1, gb2, gb3]
    LOOK = NBUF - 2

    def start_in(j):
        b = j % NBUF
        return (
            pltpu.async_copy(x_hbm.at[pl.ds(base + CH * j, CH)], xb[b], sem_in),
            pltpu.async_copy(gate_hbm.at[pl.ds(base + CH * j, CH)], gb[b], sem_in),
        )

    hin = {j: start_in(j) for j in range(min(LOOK + 1, NCH))}
    hsc = {}
    sc_waited = 0
    for j in range(NCH):
        b = j % NBUF
        for h in hin[j]:
            h.wait()
        hsc[j] = (
            pltpu.async_copy(xb[b], disp_hbm.at[idx_v.at[j]], sem_sc),
            pltpu.async_copy(gb[b], scale_hbm.at[idx_v.at[j]], sem_sc),
        )
        jn = j + LOOK + 1
        if jn < NCH:
            while sc_waited < jn - NBUF + 1:
                for h in hsc[sc_waited]:
                    h.wait()
                sc_waited += 1
            hin[jn] = start_in(jn)
    while sc_waited < NCH:
        for h in hsc[sc_waited]:
            h.wait()
        sc_waited += 1


def _dispatch(x, pos3, gate):
    mesh = plsc.VectorSubcoreMesh(core_axis_name="c", subcore_axis_name="s")
    fn = functools.partial(
        pl.kernel,
        mesh=mesh,
        out_type=[
            jax.ShapeDtypeStruct((RPAD, D_), jnp.float32),
            jax.ShapeDtypeStruct((RPAD, GW), jnp.float32),
        ],
        scratch_types=(
            [pltpu.VMEM((NCH, CH), jnp.int32)]
            + [pltpu.VMEM((CH, GW), jnp.float32) for _ in range(NBUF)]
            + [pltpu.VMEM((CH, D_), jnp.float32) for _ in range(NBUF)]
            + [pltpu.SemaphoreType.DMA, pltpu.SemaphoreType.DMA]
        ),
    )(_dispatch_body)
    return fn(x, pos3, gate)


def _mlp_body(d_ref, w1_ref, b1_ref, w2_ref, b2_ref, s_ref, o_ref):
    e = pl.program_id(0)

    @pl.when(e < E_)
    def _():
        d = d_ref[...]
        h = jnp.dot(d, w1_ref[0], preferred_element_type=jnp.float32)
        h = jnp.maximum(h + b1_ref[0], 0.0)
        o = jnp.dot(h, w2_ref[0], preferred_element_type=jnp.float32)
        o = o + b2_ref[0]
        o_ref[...] = o * s_ref[...][:, 0:1]

    @pl.when(e == E_)
    def _():
        o_ref[...] = jnp.zeros_like(o_ref)


def _mlp(disp, w1, b1, w2, b2, scale):
    ew3 = lambda e: (jnp.minimum(e, E_ - 1), 0, 0)
    return pl.pallas_call(
        _mlp_body,
        grid=(E_ + 1,),
        in_specs=[
            pl.BlockSpec((CAP, D_), lambda e: (e, 0)),
            pl.BlockSpec((1, D_, H_), ew3),
            pl.BlockSpec((1, 1, H_), ew3),
            pl.BlockSpec((1, H_, D_), ew3),
            pl.BlockSpec((1, 1, D_), ew3),
            pl.BlockSpec((CAP, GW), lambda e: (e, 0)),
        ],
        out_specs=pl.BlockSpec((CAP, D_), lambda e: (e, 0)),
        out_shape=jax.ShapeDtypeStruct((RPAD, D_), jnp.float32),
        compiler_params=pltpu.CompilerParams(
            dimension_semantics=("arbitrary",)),
    )(disp, w1, b1.reshape(E_, 1, H_), w2, b2.reshape(E_, 1, D_), scale)


def _combine_body(eo_hbm, pos_hbm, out_hbm, idx_v, buf0, buf1, buf2, buf3,
                  buf4, sem_g, sem_o):
    nc = plsc.get_sparse_core_info().num_cores
    wid = lax.axis_index("s") * nc + lax.axis_index("c")
    base = wid * TW
    pltpu.sync_copy(pos_hbm.at[wid], idx_v)

    buf = [buf0, buf1, buf2, buf3, buf4]
    LOOK = CNBUF - 2

    def start_g(j):
        return pltpu.async_copy(eo_hbm.at[idx_v.at[j]], buf[j % CNBUF], sem_g)

    hg = {j: start_g(j) for j in range(min(LOOK + 1, NCH))}
    ho = {}
    o_waited = 0
    for j in range(NCH):
        b = j % CNBUF
        hg[j].wait()
        ho[j] = pltpu.async_copy(buf[b], out_hbm.at[pl.ds(base + CH * j, CH)],
                                 sem_o)
        jn = j + LOOK + 1
        if jn < NCH:
            while o_waited < jn - CNBUF + 1:
                ho[o_waited].wait()
                o_waited += 1
            hg[jn] = start_g(jn)
    while o_waited < NCH:
        ho[o_waited].wait()
        o_waited += 1


def _combine(eo, pos3):
    mesh = plsc.VectorSubcoreMesh(core_axis_name="c", subcore_axis_name="s")
    fn = functools.partial(
        pl.kernel,
        mesh=mesh,
        out_type=jax.ShapeDtypeStruct((S_, D_), jnp.float32),
        scratch_types=(
            [pltpu.VMEM((NCH, CH), jnp.int32)]
            + [pltpu.VMEM((CH, D_), jnp.float32) for _ in range(CNBUF)]
            + [pltpu.SemaphoreType.DMA, pltpu.SemaphoreType.DMA]
        ),
    )(_combine_body)
    return fn(eo, pos3)


def kernel(hidden_states, wg, w1, b1, w2, b2):
    x = hidden_states.reshape(S_, D_)
    pos, gate, counts, laux = _gating(x, wg)
    pos3 = pos.reshape(NW, NCH, CH)
    disp, scale = _dispatch(x, pos3, gate)
    eo = _mlp(disp, w1, b1, w2, b2, scale)
    out = _combine(eo, pos3)
    return out.reshape(B_, L_, D_), laux.reshape(()), counts


# speedup vs baseline: 1.0956x; 1.0358x over previous
import functools

import jax
import jax.numpy as jnp
from jax import lax
from jax.experimental import pallas as pl
from jax.experimental.pallas import tpu as pltpu
from jax.experimental.pallas import tpu_sc as plsc

B_, L_, D_, H_, E_ = 2, 8192, 768, 768, 64
S_ = B_ * L_
CAP = 256
ZROW = E_ * CAP
RPAD = (E_ + 2) * CAP
TB = 1024
NB = S_ // TB
NW = 32
TW = S_ // NW
CH = 32
NCH = TW // CH
NBUF = 4
GW = 128
CNBUF = 5


def _gating_body(x_ref, wg_ref, pos_ref, gate_ref, cnt_ref, laux_ref,
                 cnt_acc, me_acc):
    i = pl.program_id(0)

    @pl.when(i == 0)
    def _():
        cnt_acc[...] = jnp.zeros_like(cnt_acc)
        me_acc[...] = jnp.zeros_like(me_acc)

    x = x_ref[...]
    logits = jnp.dot(x, wg_ref[...], preferred_element_type=jnp.float32)
    m = jnp.max(logits, axis=-1, keepdims=True)
    ex = jnp.exp(logits - m)
    gates = ex / jnp.sum(ex, axis=-1, keepdims=True)
    idx = jnp.argmax(gates, axis=-1).astype(jnp.int32)
    gmax = jnp.max(gates, axis=-1)

    eiota = lax.broadcasted_iota(jnp.int32, (TB, E_), 1)
    mask1 = (eiota == idx[:, None]).astype(jnp.float32)

    riota = lax.broadcasted_iota(jnp.int32, (TB, TB), 0)
    ciota = lax.broadcasted_iota(jnp.int32, (TB, TB), 1)
    tri = (ciota <= riota).astype(jnp.bfloat16)
    incl = jnp.dot(tri, mask1.astype(jnp.bfloat16),
                   preferred_element_type=jnp.float32)

    prior = cnt_acc[...]
    loc = jnp.sum((incl - 1.0 + prior) * mask1, axis=1)
    within = loc < float(CAP)
    locc = jnp.minimum(loc, float(CAP - 1)).astype(jnp.int32)
    pos = idx * CAP + locc
    pos_ref[...] = jnp.where(within, pos, ZROW)
    gate_ref[...] = jnp.broadcast_to(gmax[:, None], (TB, GW))

    cnt_new = prior + jnp.sum(mask1, axis=0, keepdims=True)
    cnt_acc[...] = cnt_new
    me_new = me_acc[...] + jnp.sum(gates, axis=0, keepdims=True)
    me_acc[...] = me_new

    @pl.when(i == NB - 1)
    def _():
        cnt_ref[...] = cnt_new[0].astype(jnp.int32)
        me = me_new / float(S_)
        ce = cnt_new / float(S_)
        laux_ref[...] = jnp.sum(me * ce).reshape(1, 1) * float(E_)


def _gating(x, wg):
    return pl.pallas_call(
        _gating_body,
        grid=(NB,),
        in_specs=[
            pl.BlockSpec((TB, D_), lambda i: (i, 0)),
            pl.BlockSpec((D_, E_), lambda i: (0, 0)),
        ],
        out_specs=[
            pl.BlockSpec((TB,), lambda i: (i,)),
            pl.BlockSpec((TB, GW), lambda i: (i, 0)),
            pl.BlockSpec((E_,), lambda i: (0,)),
            pl.BlockSpec((1, 1), lambda i: (0, 0)),
        ],
        out_shape=[
            jax.ShapeDtypeStruct((S_,), jnp.int32),
            jax.ShapeDtypeStruct((S_, GW), jnp.float32),
            jax.ShapeDtypeStruct((E_,), jnp.int32),
            jax.ShapeDtypeStruct((1, 1), jnp.float32),
        ],
        scratch_shapes=[
            pltpu.VMEM((1, E_), jnp.float32),
            pltpu.VMEM((1, E_), jnp.float32),
        ],
        compiler_params=pltpu.CompilerParams(
            dimension_semantics=("arbitrary",)),
    )(x, wg)


def _dispatch_body(x_hbm, pos_hbm, gate_hbm, disp_hbm, scale_hbm,
                   idx_v, gb0, gb1, gb2, gb3, xb0, xb1, xb2, xb3,
                   sem_in, sem_sc):
    nc = plsc.get_sparse_core_info().num_cores
    wid = lax.axis_index("s") * nc + lax.axis_index("c")
    base = wid * TW
    pltpu.sync_copy(pos_hbm.at[wid], idx_v)

    xb = [xb0, xb1, xb2, xb3]
    gb = [gb0, gb1, gb2, gb3]
    LOOK = NBUF - 2

    def start_in(j):
        b = j % NBUF
        return (
            pltpu.async_copy(x_hbm.at[pl.ds(base + CH * j, CH)], xb[b], sem_in),
            pltpu.async_copy(gate_hbm.at[pl.ds(base + CH * j, CH)], gb[b], sem_in),
        )

    hin = {j: start_in(j) for j in range(min(LOOK + 1, NCH))}
    hsc = {}
    sc_waited = 0
    for j in range(NCH):
        b = j % NBUF
        for h in hin[j]:
            h.wait()
        hsc[j] = (
            pltpu.async_copy(xb[b], disp_hbm.at[idx_v.at[j]], sem_sc),
            pltpu.async_copy(gb[b], scale_hbm.at[idx_v.at[j]], sem_sc),
        )
        jn = j + LOOK + 1
        if jn < NCH:
            while sc_waited < jn - NBUF + 1:
                for h in hsc[sc_waited]:
                    h.wait()
                sc_waited += 1
            hin[jn] = start_in(jn)
    while sc_waited < NCH:
        for h in hsc[sc_waited]:
            h.wait()
        sc_waited += 1


def _dispatch(x, pos3, gate):
    mesh = plsc.VectorSubcoreMesh(core_axis_name="c", subcore_axis_name="s")
    fn = functools.partial(
        pl.kernel,
        mesh=mesh,
        out_type=[
            jax.ShapeDtypeStruct((RPAD, D_), jnp.float32),
            jax.ShapeDtypeStruct((RPAD, GW), jnp.float32),
        ],
        scratch_types=(
            [pltpu.VMEM((NCH, CH), jnp.int32)]
            + [pltpu.VMEM((CH, GW), jnp.float32) for _ in range(NBUF)]
            + [pltpu.VMEM((CH, D_), jnp.float32) for _ in range(NBUF)]
            + [pltpu.SemaphoreType.DMA, pltpu.SemaphoreType.DMA]
        ),
    )(_dispatch_body)
    return fn(x, pos3, gate)


def _mlp_body(d_ref, w1_ref, b1_ref, w2_ref, b2_ref, s_ref, o_ref):
    e = pl.program_id(0)

    @pl.when(e < E_ // 2)
    def _():
        sv = s_ref[...]
        for k in range(2):
            d = d_ref[k * CAP:(k + 1) * CAP, :]
            h = jnp.dot(d, w1_ref[k], preferred_element_type=jnp.float32)
            h = jnp.maximum(h + b1_ref[k], 0.0)
            o = jnp.dot(h, w2_ref[k], preferred_element_type=jnp.float32)
            o = o + b2_ref[k]
            o_ref[k * CAP:(k + 1) * CAP, :] = o * sv[k * CAP:(k + 1) * CAP, 0:1]

    @pl.when(e == E_ // 2)
    def _():
        o_ref[...] = jnp.zeros_like(o_ref)


def _mlp(disp, w1, b1, w2, b2, scale):
    ew3 = lambda e: (jnp.minimum(e, E_ // 2 - 1), 0, 0)
    return pl.pallas_call(
        _mlp_body,
        grid=(E_ // 2 + 1,),
        in_specs=[
            pl.BlockSpec((2 * CAP, D_), lambda e: (e, 0)),
            pl.BlockSpec((2, D_, H_), ew3),
            pl.BlockSpec((2, 1, H_), ew3),
            pl.BlockSpec((2, H_, D_), ew3),
            pl.BlockSpec((2, 1, D_), ew3),
            pl.BlockSpec((2 * CAP, GW), lambda e: (e, 0)),
        ],
        out_specs=pl.BlockSpec((2 * CAP, D_), lambda e: (e, 0)),
        out_shape=jax.ShapeDtypeStruct((RPAD, D_), jnp.float32),
        compiler_params=pltpu.CompilerParams(
            dimension_semantics=("arbitrary",)),
    )(disp, w1, b1.reshape(E_, 1, H_), w2, b2.reshape(E_, 1, D_), scale)


def _combine_body(eo_hbm, pos_hbm, out_hbm, idx_v, buf0, buf1, buf2, buf3,
                  buf4, sem_g, sem_o):
    nc = plsc.get_sparse_core_info().num_cores
    wid = lax.axis_index("s") * nc + lax.axis_index("c")
    base = wid * TW
    pltpu.sync_copy(pos_hbm.at[wid], idx_v)

    buf = [buf0, buf1, buf2, buf3, buf4]
    LOOK = CNBUF - 2

    def start_g(j):
        return pltpu.async_copy(eo_hbm.at[idx_v.at[j]], buf[j % CNBUF], sem_g)

    hg = {j: start_g(j) for j in range(min(LOOK + 1, NCH))}
    ho = {}
    o_waited = 0
    for j in range(NCH):
        b = j % CNBUF
        hg[j].wait()
        ho[j] = pltpu.async_copy(buf[b], out_hbm.at[pl.ds(base + CH * j, CH)],
                                 sem_o)
        jn = j + LOOK + 1
        if jn < NCH:
            while o_waited < jn - CNBUF + 1:
                ho[o_waited].wait()
                o_waited += 1
            hg[jn] = start_g(jn)
    while o_waited < NCH:
        ho[o_waited].wait()
        o_waited += 1


def _combine(eo, pos3):
    mesh = plsc.VectorSubcoreMesh(core_axis_name="c", subcore_axis_name="s")
    fn = functools.partial(
        pl.kernel,
        mesh=mesh,
        out_type=jax.ShapeDtypeStruct((S_, D_), jnp.float32),
        scratch_types=(
            [pltpu.VMEM((NCH, CH), jnp.int32)]
            + [pltpu.VMEM((CH, D_), jnp.float32) for _ in range(CNBUF)]
            + [pltpu.SemaphoreType.DMA, pltpu.SemaphoreType.DMA]
        ),
    )(_combine_body)
    return fn(eo, pos3)


def kernel(hidden_states, wg, w1, b1, w2, b2):
    x = hidden_states.reshape(S_, D_)
    pos, gate, counts, laux = _gating(x, wg)
    pos3 = pos.reshape(NW, NCH, CH)
    disp, scale = _dispatch(x, pos3, gate)
    eo = _mlp(disp, w1, b1, w2, b2, scale)
    out = _combine(eo, pos3)
    return out.reshape(B_, L_, D_), laux.reshape(()), counts
